# initial kernel scaffold (unmeasured)
import jax
import jax.numpy as jnp
from jax import lax
from jax.experimental import pallas as pl
from jax.experimental.pallas import tpu as pltpu


def kernel(
    x,
):
    def body(*refs):
        pass

    out_shape = jax.ShapeDtypeStruct(..., jnp.float32)
    return pl.pallas_call(body, out_shape=out_shape)(...)


# baseline (device time: 23722 ns/iter reference)
import jax
import jax.numpy as jnp
from jax import lax
from jax.experimental import pallas as pl
from jax.experimental.pallas import tpu as pltpu

N_DEV = 16


def kernel(x):
    m_per, n = x.shape

    def body(x_ref, out_ref, comm_ref, send_sems, recv_sems):
        my_pos = lax.axis_index("i")

        xv = x_ref[...]
        val = jnp.max(xv, axis=0)
        rows = lax.broadcasted_iota(jnp.int32, (m_per, n), 0)
        cand = jnp.where(xv == val[None, :], rows, jnp.int32(m_per))
        lidx = jnp.min(cand, axis=0)
        gidx = (lidx + my_pos * m_per).astype(jnp.float32)

        comm_ref[0, 0, :] = val
        comm_ref[0, 1, :] = gidx

        rdmas = []
        for d in range(1, N_DEV):
            tgt = (my_pos + d) % N_DEV
            rdma = pltpu.make_async_remote_copy(
                src_ref=comm_ref.at[0],
                dst_ref=comm_ref.at[d],
                send_sem=send_sems.at[d - 1],
                recv_sem=recv_sems.at[d - 1],
                device_id=(tgt,),
                device_id_type=pl.DeviceIdType.MESH,
            )
            rdma.start()
            rdmas.append(rdma)
        for r in rdmas:
            r.wait_send()
        for r in rdmas:
            r.wait_recv()

        allp = comm_ref[...]
        vals = allp[:, 0, :]
        idxs = allp[:, 1, :]
        gmax = jnp.max(vals, axis=0)
        big = jnp.float32(N_DEV * m_per + 1)
        candi = jnp.where(vals == gmax[None, :], idxs, big)
        out_ref[0, :] = gmax
        out_ref[1, :] = jnp.min(candi, axis=0)

    return pl.pallas_call(
        body,
        out_shape=jax.ShapeDtypeStruct((2, n), jnp.float32),
        in_specs=[pl.BlockSpec(memory_space=pltpu.VMEM)],
        out_specs=pl.BlockSpec(memory_space=pltpu.VMEM),
        scratch_shapes=[
            pltpu.VMEM((N_DEV, 2, n), jnp.float32),
            pltpu.SemaphoreType.DMA((N_DEV - 1,)),
            pltpu.SemaphoreType.DMA((N_DEV - 1,)),
        ],
    )(x)


# device time: 18773 ns/iter; 1.2636x vs baseline; 1.2636x over previous
import jax
import jax.numpy as jnp
from jax import lax
from jax.experimental import pallas as pl
from jax.experimental.pallas import tpu as pltpu

N_DEV = 16
BM = 512


def kernel(x):
    m_per, n = x.shape
    nblk = m_per // BM

    def body(x_ref, out_ref, acc_val, acc_idx, comm_ref, send_sems, recv_sems):
        b = pl.program_id(0)
        my_pos = lax.axis_index("i")

        xv = x_ref[...]
        bval = jnp.max(xv, axis=0)
        rows = lax.broadcasted_iota(jnp.int32, (BM, n), 0)
        cand = jnp.where(xv == bval[None, :], rows, jnp.int32(BM))
        bidx = jnp.min(cand, axis=0) + b * BM

        @pl.when(b == 0)
        def _():
            acc_val[0, :] = bval
            acc_idx[0, :] = bidx

        @pl.when(b > 0)
        def _():
            better = bval > acc_val[0, :]
            acc_val[0, :] = jnp.where(better, bval, acc_val[0, :])
            acc_idx[0, :] = jnp.where(better, bidx, acc_idx[0, :])

        @pl.when(b == nblk - 1)
        def _():
            comm_ref[0, 0, :] = acc_val[0, :]
            comm_ref[0, 1, :] = (acc_idx[0, :] + my_pos * m_per).astype(
                jnp.float32
            )

            barrier_sem = pltpu.get_barrier_semaphore()
            for d in range(1, N_DEV):
                pl.semaphore_signal(
                    barrier_sem,
                    inc=1,
                    device_id=((my_pos + d) % N_DEV,),
                    device_id_type=pl.DeviceIdType.MESH,
                )
            pl.semaphore_wait(barrier_sem, N_DEV - 1)

            rdmas = []
            for d in range(1, N_DEV):
                rdma = pltpu.make_async_remote_copy(
                    src_ref=comm_ref.at[0],
                    dst_ref=comm_ref.at[d],
                    send_sem=send_sems.at[d - 1],
                    recv_sem=recv_sems.at[d - 1],
                    device_id=((my_pos + d) % N_DEV,),
                    device_id_type=pl.DeviceIdType.MESH,
                )
                rdma.start()
                rdmas.append(rdma)
            for r in rdmas:
                r.wait_send()
            for r in rdmas:
                r.wait_recv()

            allp = comm_ref[...]
            vals = allp[:, 0, :]
            idxs = allp[:, 1, :]
            gmax = jnp.max(vals, axis=0)
            big = jnp.float32(N_DEV * m_per + 1)
            candi = jnp.where(vals == gmax[None, :], idxs, big)
            out_ref[0, :] = gmax
            out_ref[1, :] = jnp.min(candi, axis=0)

    return pl.pallas_call(
        body,
        grid=(nblk,),
        out_shape=jax.ShapeDtypeStruct((2, n), jnp.float32),
        in_specs=[pl.BlockSpec((BM, n), lambda b: (b, 0))],
        out_specs=pl.BlockSpec((2, n), lambda b: (0, 0)),
        scratch_shapes=[
            pltpu.VMEM((1, n), jnp.float32),
            pltpu.VMEM((1, n), jnp.int32),
            pltpu.VMEM((N_DEV, 2, n), jnp.float32),
            pltpu.SemaphoreType.DMA((N_DEV - 1,)),
            pltpu.SemaphoreType.DMA((N_DEV - 1,)),
        ],
        compiler_params=pltpu.CompilerParams(collective_id=0),
    )(x)


# device time: 16803 ns/iter; 1.4118x vs baseline; 1.1172x over previous
import jax
import jax.numpy as jnp
from jax import lax
from jax.experimental import pallas as pl
from jax.experimental.pallas import tpu as pltpu

N_DEV = 16
BM = 1024


def kernel(x):
    m_per, n = x.shape
    nblk = m_per // BM

    def body(x_ref, out_ref, acc_val, acc_idx, comm_ref, send_sems, recv_sems):
        b = pl.program_id(0)
        my_pos = lax.axis_index("i")
        barrier_sem = pltpu.get_barrier_semaphore()

        @pl.when(b == 0)
        def _():
            for d in range(1, N_DEV):
                pl.semaphore_signal(
                    barrier_sem,
                    inc=1,
                    device_id=((my_pos + d) % N_DEV,),
                    device_id_type=pl.DeviceIdType.MESH,
                )

        xv = x_ref[...]
        bval = jnp.max(xv, axis=0)
        bidx = jnp.argmax(xv, axis=0).astype(jnp.int32) + b * BM

        @pl.when(b == 0)
        def _():
            acc_val[0, :] = bval
            acc_idx[0, :] = bidx

        @pl.when(b > 0)
        def _():
            better = bval > acc_val[0, :]
            acc_val[0, :] = jnp.where(better, bval, acc_val[0, :])
            acc_idx[0, :] = jnp.where(better, bidx, acc_idx[0, :])

        @pl.when(b == nblk - 1)
        def _():
            comm_ref[0, 0, :] = acc_val[0, :]
            comm_ref[0, 1, :] = (acc_idx[0, :] + my_pos * m_per).astype(
                jnp.float32
            )

            pl.semaphore_wait(barrier_sem, N_DEV - 1)

            rdmas = []
            for d in range(1, N_DEV):
                rdma = pltpu.make_async_remote_copy(
                    src_ref=comm_ref.at[0],
                    dst_ref=comm_ref.at[d],
                    send_sem=send_sems.at[d - 1],
                    recv_sem=recv_sems.at[d - 1],
                    device_id=((my_pos + d) % N_DEV,),
                    device_id_type=pl.DeviceIdType.MESH,
                )
                rdma.start()
                rdmas.append(rdma)
            for r in rdmas:
                r.wait_send()
            for r in rdmas:
                r.wait_recv()

            allp = comm_ref[...]
            vals = allp[:, 0, :]
            idxs = allp[:, 1, :]
            gmax = jnp.max(vals, axis=0)
            big = jnp.float32(N_DEV * m_per + 1)
            candi = jnp.where(vals == gmax[None, :], idxs, big)
            out_ref[0, :] = gmax
            out_ref[1, :] = jnp.min(candi, axis=0)

    return pl.pallas_call(
        body,
        grid=(nblk,),
        out_shape=jax.ShapeDtypeStruct((2, n), jnp.float32),
        in_specs=[pl.BlockSpec((BM, n), lambda b: (b, 0))],
        out_specs=pl.BlockSpec((2, n), lambda b: (0, 0)),
        scratch_shapes=[
            pltpu.VMEM((1, n), jnp.float32),
            pltpu.VMEM((1, n), jnp.int32),
            pltpu.VMEM((N_DEV, 2, n), jnp.float32),
            pltpu.SemaphoreType.DMA((N_DEV - 1,)),
            pltpu.SemaphoreType.DMA((N_DEV - 1,)),
        ],
        compiler_params=pltpu.CompilerParams(collective_id=0),
    )(x)
